# baseline (speedup 1.0000x reference)
import functools

import jax
import jax.numpy as jnp
import numpy as np
from jax import lax
from jax.experimental import pallas as pl
from jax.experimental.pallas import tpu as pltpu
from jax.experimental.pallas import tpu_sc as plsc

_N = 100000
_E = 1600000
_H = 128
_EP = 1638400
_NACC = 102400
_B = 1024
_NT = 16
_NC = 2
_BN = 2000
_NP = 100096
_N2 = 100096
_ZR = 400


def _make_sc_agg(l1: bool):
    rounds = 1 if l1 else 4
    nslots = 2 if l1 else 8
    per_core = _EP // 2 if l1 else _EP
    per_tile = per_core // _NT
    n_batches = per_tile // _B
    rows_per_tile = _NACC // _NT
    nzero = rows_per_tile // _ZR

    mesh = plsc.VectorSubcoreMesh(core_axis_name="c", subcore_axis_name="s")

    @functools.partial(
        pl.kernel,
        out_type=jax.ShapeDtypeStruct((nslots, _NACC, 16), jnp.float32),
        mesh=mesh,
        scratch_types=[
            pltpu.VMEM((_B,), jnp.int32),
            pltpu.VMEM((_B,), jnp.int32),
            pltpu.VMEM((_B // 128, 128), jnp.int32),
            pltpu.VMEM((_B // 128, 128), jnp.int32),
            pltpu.VMEM((_B, 16), jnp.float32),
            pltpu.VMEM((_ZR, 16), jnp.float32),
            pltpu.VMEM_SHARED((_NACC, 16), jnp.float32),
            pltpu.SemaphoreType.DMA,
            pltpu.SemaphoreType.DMA,
            pltpu.SemaphoreType.DMA,
        ],
        compiler_params=pltpu.CompilerParams(use_tc_tiling_on_sc=False),
    )
    def k(table, srcp, dstp, out, src_a, src_b, dst_a, dst_b, rows_v, zbuf,
          acc, sem_i, sem_g, sem_s):
        c = lax.axis_index("c")
        s = lax.axis_index("s")
        row0 = pl.multiple_of(s * rows_per_tile, 128)
        idx_bufs = ((src_a, dst_a), (src_b, dst_b))

        def zb(i, carry):
            zbuf[i] = jnp.zeros((16,), jnp.float32)
            return carry
        lax.fori_loop(0, _ZR, zb, 0)

        for r in range(rounds):
            if l1:
                slot = c
                ebase0 = c * per_core + s * per_tile
                tbl = table
            else:
                slot = r * _NC + c
                ebase0 = s * per_tile
                tbl = table.at[pl.ds(pl.multiple_of(slot * _N2, 128), _N2)]
            rbase0 = ebase0 // 128

            def idx_load(b, bufs):
                eb = pl.multiple_of(ebase0 + b * _B, 128)
                rb = pl.multiple_of(rbase0 + b * (_B // 128), 8)
                return (
                    pltpu.async_copy(srcp.at[pl.ds(eb, _B)], bufs[0], sem_i),
                    pltpu.async_copy(dstp.at[pl.ds(rb, _B // 128)], bufs[1],
                                     sem_i),
                )

            idx_load(0, idx_bufs[0])
            zd = [pltpu.async_copy(zbuf, acc.at[pl.ds(row0 + z * _ZR, _ZR)],
                                   sem_g)
                  for z in range(nzero)]
            for d in zd:
                d.wait()
            plsc.subcore_barrier()

            nj = _B // 128
            nh = nj // 2

            def scatter_drain(dst_v, h):
                for j in range(h * nh, (h + 1) * nh):
                    pltpu.make_async_copy(
                        rows_v.at[pl.ds(j * 128, 128)],
                        acc.at[dst_v.at[j]], sem_s).wait()

            def pair(i, carry):
                for p in range(2):
                    b = 2 * i + p
                    src_v, dst_v = idx_bufs[p]
                    pltpu.make_async_copy(srcp.at[pl.ds(0, _B)], src_v,
                                          sem_i).wait()
                    pltpu.make_async_copy(dstp.at[pl.ds(0, _B // 128)], dst_v,
                                          sem_i).wait()
                    gd = []
                    for h in range(2):
                        @pl.when(b > 0)
                        def _(h=h):
                            scatter_drain(idx_bufs[1 - p][1], h)
                        gd.append([
                            pltpu.async_copy(
                                tbl.at[src_v.at[pl.ds(j * 128, 128)]],
                                rows_v.at[pl.ds(j * 128, 128)], sem_g)
                            for j in range(h * nh, (h + 1) * nh)
                        ])
                    idx_load(b + 1, idx_bufs[1 - p])
                    for h in range(2):
                        for d in gd[h]:
                            d.wait()
                        for j in range(h * nh, (h + 1) * nh):
                            pltpu.async_copy(rows_v.at[pl.ds(j * 128, 128)],
                                             acc.at[dst_v.at[j]], sem_s,
                                             add=True)
                return carry
            lax.fori_loop(0, n_batches // 2, pair, 0)
            scatter_drain(idx_bufs[1][1], 0)
            scatter_drain(idx_bufs[1][1], 1)
            pltpu.make_async_copy(srcp.at[pl.ds(0, _B)], src_a, sem_i).wait()
            pltpu.make_async_copy(dstp.at[pl.ds(0, _B // 128)], dst_a,
                                  sem_i).wait()
            plsc.subcore_barrier()

            pltpu.sync_copy(acc.at[pl.ds(row0, rows_per_tile)],
                            out.at[slot, pl.ds(row0, rows_per_tile)])

    return k


_sc_l1 = _make_sc_agg(l1=True)
_sc_l2 = _make_sc_agg(l1=False)


def _norm_body(xt_ref, cs_ref, o_ref):
    neg = jnp.float32(-3.0e38)
    col = lax.broadcasted_iota(jnp.int32, (1, _N2), 1)
    valid = col < _N
    x0 = xt_ref[0:1, :]
    x1 = xt_ref[1:2, :]
    x2 = xt_ref[2:3, :]

    def vmax(v):
        return jnp.max(jnp.where(valid, v, neg))

    def vmin(v):
        return jnp.min(jnp.where(valid, v, -neg))

    def vsum(v):
        return jnp.sum(jnp.where(valid, v, 0.0))

    max0, min0 = vmax(x0), vmin(x0)
    max1, min1 = vmax(x1), vmin(x1)
    ct = cs_ref[0, 0]
    st = cs_ref[0, 1]
    cond = (max1 - min1) > (max0 - min0)
    rm = jnp.stack([jnp.stack([ct, -st]), jnp.stack([st, ct])])
    rot = jnp.dot(rm, xt_ref[0:2, :], preferred_element_type=jnp.float32)
    sel0 = jnp.where(cond, rot[0:1, :], x0)
    sel1 = jnp.where(cond, rot[1:2, :], x1)
    m0 = vsum(sel0) / jnp.float32(_N)
    m1 = vsum(sel1) / jnp.float32(_N)
    mx0 = vmax(sel0)
    mx1 = vmax(sel1)
    amax = vmax(x2)
    c0 = (sel0 - m0) / mx0
    c1 = (sel1 - m1) / mx1
    an = x2 / amax
    one = jnp.ones_like(an)
    zer = jnp.zeros((12, _N2), jnp.float32)
    o_ref[...] = jnp.concatenate([c0, c1, an, one, zer], axis=0)


_norm_call = pl.pallas_call(
    _norm_body,
    out_shape=jax.ShapeDtypeStruct((16, _N2), jnp.float32),
)


def _l1_body(a_ref, xn_ref, wl_ref, wr_ref, b_ref, hc_ref, cnt_ref):
    a = a_ref[0] + a_ref[1]
    cnt = a[:, 3:4]
    mean = a / jnp.maximum(cnt, 1.0)
    h = jnp.dot(mean, wl_ref[...], preferred_element_type=jnp.float32)
    h = h + jnp.dot(xn_ref[...], wr_ref[...], preferred_element_type=jnp.float32)
    h = h + b_ref[...]
    h = jnp.tanh(h)
    for j in range(8):
        hc_ref[j] = h[:, j * 16:(j + 1) * 16]
    cnt_ref[...] = cnt


_l1_call = pl.pallas_call(
    _l1_body,
    grid=(_N // _BN,),
    in_specs=[
        pl.BlockSpec((2, _BN, 16), lambda i: (0, i, 0)),
        pl.BlockSpec((_BN, 16), lambda i: (i, 0)),
        pl.BlockSpec((16, _H), lambda i: (0, 0)),
        pl.BlockSpec((16, _H), lambda i: (0, 0)),
        pl.BlockSpec((1, _H), lambda i: (0, 0)),
    ],
    out_specs=[
        pl.BlockSpec((8, _BN, 16), lambda i: (0, i, 0)),
        pl.BlockSpec((_BN, 1), lambda i: (i, 0)),
    ],
    out_shape=[
        jax.ShapeDtypeStruct((8, _N2, 16), jnp.float32),
        jax.ShapeDtypeStruct((_N, 1), jnp.float32),
    ],
)


def _final_body(a2_ref, cnt_ref, hc_ref, wl2_ref, wr2_ref, bl2_ref,
                wlin_ref, blin_ref, wlast_ref, blast_ref, o_ref):
    summed = jnp.concatenate([a2_ref[j] for j in range(8)], axis=1)
    h1 = jnp.concatenate([hc_ref[j] for j in range(8)], axis=1)
    mean = summed / jnp.maximum(cnt_ref[...], 1.0)
    h2 = jnp.tanh(
        jnp.dot(mean, wl2_ref[...], preferred_element_type=jnp.float32)
        + jnp.dot(h1, wr2_ref[...], preferred_element_type=jnp.float32)
        + bl2_ref[...])
    h3 = jnp.tanh(
        jnp.dot(h2, wlin_ref[...], preferred_element_type=jnp.float32)
        + blin_ref[...])
    lg = jnp.dot(h3, wlast_ref[...], preferred_element_type=jnp.float32)
    lg = lg + blast_ref[...]
    m = jnp.max(lg, axis=1, keepdims=True)
    e = jnp.exp(lg - m)
    o_ref[...] = e / jnp.sum(e, axis=1, keepdims=True)


_final_call = pl.pallas_call(
    _final_body,
    grid=(_N // _BN,),
    in_specs=[
        pl.BlockSpec((8, _BN, 16), lambda i: (0, i, 0)),
        pl.BlockSpec((_BN, 1), lambda i: (i, 0)),
        pl.BlockSpec((8, _BN, 16), lambda i: (0, i, 0)),
        pl.BlockSpec((_H, _H), lambda i: (0, 0)),
        pl.BlockSpec((_H, _H), lambda i: (0, 0)),
        pl.BlockSpec((1, _H), lambda i: (0, 0)),
        pl.BlockSpec((_H, _H), lambda i: (0, 0)),
        pl.BlockSpec((1, _H), lambda i: (0, 0)),
        pl.BlockSpec((_H, 4), lambda i: (0, 0)),
        pl.BlockSpec((1, 4), lambda i: (0, 0)),
    ],
    out_specs=pl.BlockSpec((_BN, 4), lambda i: (i, 0)),
    out_shape=jax.ShapeDtypeStruct((_N, 4), jnp.float32),
)


def kernel(x, edge_index, Wl1, bl1, Wr1, Wl2, bl2, Wr2,
           W_lin1, b_lin1, W_last, b_last):
    src = edge_index[0]
    dst = edge_index[1]
    pad = _EP + _B - _E
    srcp = jnp.concatenate([src, jnp.zeros((pad,), jnp.int32)])
    dstp = jnp.concatenate([dst, jnp.full((pad,), _N, jnp.int32)])
    dstp = dstp.reshape((_EP + _B) // 128, 128)

    xtp = jnp.pad(x.T, ((0, 0), (0, _N2 - _N)))
    theta = jnp.float32(np.pi / 2)
    ctst = jnp.stack([jnp.cos(theta), jnp.sin(theta)]).reshape(1, 2)
    xn16 = _norm_call(xtp, ctst).T

    agg1 = _sc_l1(xn16, srcp, dstp)
    Wl1p = jnp.pad(Wl1, ((0, 13), (0, 0)))
    Wr1p = jnp.pad(Wr1, ((0, 13), (0, 0)))
    h1c, cnt = _l1_call(agg1, xn16, Wl1p, Wr1p, bl1.reshape(1, _H))

    agg2 = _sc_l2(h1c.reshape(8 * _N2, 16), srcp, dstp)
    out = _final_call(agg2, cnt, h1c, Wl2, Wr2, bl2.reshape(1, _H),
                      W_lin1, b_lin1.reshape(1, _H),
                      W_last, b_last.reshape(1, 4))
    return out


# speedup vs baseline: 1.0224x; 1.0224x over previous
import functools

import jax
import jax.numpy as jnp
import numpy as np
from jax import lax
from jax.experimental import pallas as pl
from jax.experimental.pallas import tpu as pltpu
from jax.experimental.pallas import tpu_sc as plsc

_N = 100000
_E = 1600000
_H = 128
_EP = 1638400
_NACC = 102400
_B = 1024
_NT = 16
_NC = 2
_BN = 2000
_NP = 100096
_N2 = 100096
_ZR = 400


def _make_sc_agg(l1: bool):
    rounds = 1 if l1 else 4
    nslots = 2 if l1 else 8
    per_core = _EP // 2 if l1 else _EP
    per_tile = per_core // _NT
    n_batches = per_tile // _B
    rows_per_tile = _NACC // _NT
    nzero = rows_per_tile // _ZR

    mesh = plsc.VectorSubcoreMesh(core_axis_name="c", subcore_axis_name="s")

    @functools.partial(
        pl.kernel,
        out_type=jax.ShapeDtypeStruct((nslots, _NACC, 16), jnp.float32),
        mesh=mesh,
        scratch_types=[
            pltpu.VMEM((_B,), jnp.int32),
            pltpu.VMEM((_B,), jnp.int32),
            pltpu.VMEM((_B // 128, 128), jnp.int32),
            pltpu.VMEM((_B // 128, 128), jnp.int32),
            pltpu.VMEM((_B, 16), jnp.float32),
            pltpu.VMEM((_ZR, 16), jnp.float32),
            pltpu.VMEM_SHARED((_NACC, 16), jnp.float32),
            pltpu.SemaphoreType.DMA,
            pltpu.SemaphoreType.DMA,
            pltpu.SemaphoreType.DMA,
        ],
        compiler_params=pltpu.CompilerParams(use_tc_tiling_on_sc=False),
    )
    def k(table, srcp, dstp, out, src_a, src_b, dst_a, dst_b, rows_v, zbuf,
          acc, sem_i, sem_g, sem_s):
        c = lax.axis_index("c")
        s = lax.axis_index("s")
        row0 = pl.multiple_of(s * rows_per_tile, 128)
        idx_bufs = ((src_a, dst_a), (src_b, dst_b))

        def zb(i, carry):
            zbuf[i] = jnp.zeros((16,), jnp.float32)
            return carry
        lax.fori_loop(0, _ZR, zb, 0)

        for r in range(rounds):
            if l1:
                slot = c
                ebase0 = c * per_core + s * per_tile
                tbl = table
            else:
                slot = r * _NC + c
                ebase0 = s * per_tile
                tbl = table.at[pl.ds(pl.multiple_of(slot * _N2, 128), _N2)]
            rbase0 = ebase0 // 128

            def idx_load(b, bufs):
                eb = pl.multiple_of(ebase0 + b * _B, 128)
                rb = pl.multiple_of(rbase0 + b * (_B // 128), 8)
                return (
                    pltpu.async_copy(srcp.at[pl.ds(eb, _B)], bufs[0], sem_i),
                    pltpu.async_copy(dstp.at[pl.ds(rb, _B // 128)], bufs[1],
                                     sem_i),
                )

            idx_load(0, idx_bufs[0])
            zd = [pltpu.async_copy(zbuf, acc.at[pl.ds(row0 + z * _ZR, _ZR)],
                                   sem_g)
                  for z in range(nzero)]
            for d in zd:
                d.wait()
            plsc.subcore_barrier()

            nj = _B // 128
            nh = nj // 2

            def scatter_drain(dst_v, h):
                for j in range(h * nh, (h + 1) * nh):
                    pltpu.make_async_copy(
                        rows_v.at[pl.ds(j * 128, 128)],
                        acc.at[dst_v.at[j]], sem_s).wait()

            def pair(i, carry):
                for p in range(2):
                    b = 2 * i + p
                    src_v, dst_v = idx_bufs[p]
                    pltpu.make_async_copy(srcp.at[pl.ds(0, _B)], src_v,
                                          sem_i).wait()
                    pltpu.make_async_copy(dstp.at[pl.ds(0, _B // 128)], dst_v,
                                          sem_i).wait()
                    gd = []
                    for h in range(2):
                        @pl.when(b > 0)
                        def _(h=h):
                            scatter_drain(idx_bufs[1 - p][1], h)
                        gd.append([
                            pltpu.async_copy(
                                tbl.at[src_v.at[pl.ds(j * 128, 128)]],
                                rows_v.at[pl.ds(j * 128, 128)], sem_g)
                            for j in range(h * nh, (h + 1) * nh)
                        ])
                    idx_load(b + 1, idx_bufs[1 - p])
                    for h in range(2):
                        for d in gd[h]:
                            d.wait()
                        for j in range(h * nh, (h + 1) * nh):
                            pltpu.async_copy(rows_v.at[pl.ds(j * 128, 128)],
                                             acc.at[dst_v.at[j]], sem_s,
                                             add=True)
                return carry
            lax.fori_loop(0, n_batches // 2, pair, 0)
            scatter_drain(idx_bufs[1][1], 0)
            scatter_drain(idx_bufs[1][1], 1)
            pltpu.make_async_copy(srcp.at[pl.ds(0, _B)], src_a, sem_i).wait()
            pltpu.make_async_copy(dstp.at[pl.ds(0, _B // 128)], dst_a,
                                  sem_i).wait()
            plsc.subcore_barrier()

            pltpu.sync_copy(acc.at[pl.ds(row0, rows_per_tile)],
                            out.at[slot, pl.ds(row0, rows_per_tile)])

    return k


_sc_l1 = _make_sc_agg(l1=True)
_sc_l2 = _make_sc_agg(l1=False)


def _norm_body(xt_ref, cs_ref, o_ref):
    neg = jnp.float32(-3.0e38)
    col = lax.broadcasted_iota(jnp.int32, (1, _N2), 1)
    valid = col < _N
    x0 = xt_ref[0:1, :]
    x1 = xt_ref[1:2, :]
    x2 = xt_ref[2:3, :]

    def vmax(v):
        return jnp.max(jnp.where(valid, v, neg))

    def vmin(v):
        return jnp.min(jnp.where(valid, v, -neg))

    def vsum(v):
        return jnp.sum(jnp.where(valid, v, 0.0))

    max0, min0 = vmax(x0), vmin(x0)
    max1, min1 = vmax(x1), vmin(x1)
    ct = cs_ref[0, 0]
    st = cs_ref[0, 1]
    cond = (max1 - min1) > (max0 - min0)
    rm = jnp.stack([jnp.stack([ct, -st]), jnp.stack([st, ct])])
    rot = jnp.dot(rm, xt_ref[0:2, :], preferred_element_type=jnp.float32)
    sel0 = jnp.where(cond, rot[0:1, :], x0)
    sel1 = jnp.where(cond, rot[1:2, :], x1)
    m0 = vsum(sel0) / jnp.float32(_N)
    m1 = vsum(sel1) / jnp.float32(_N)
    mx0 = vmax(sel0)
    mx1 = vmax(sel1)
    amax = vmax(x2)
    c0 = (sel0 - m0) / mx0
    c1 = (sel1 - m1) / mx1
    an = x2 / amax
    one = jnp.ones_like(an)
    zer = jnp.zeros((12, _N2), jnp.float32)
    o_ref[...] = jnp.concatenate([c0, c1, an, one, zer], axis=0)


_norm_call = pl.pallas_call(
    _norm_body,
    out_shape=jax.ShapeDtypeStruct((16, _N2), jnp.float32),
)


def _l1_body(a_ref, xn_ref, wl_ref, wr_ref, b_ref, h_ref, hc_ref, cnt_ref):
    a = a_ref[0] + a_ref[1]
    cnt = a[:, 3:4]
    mean = a / jnp.maximum(cnt, 1.0)
    h = jnp.dot(mean, wl_ref[...], preferred_element_type=jnp.float32)
    h = h + jnp.dot(xn_ref[...], wr_ref[...], preferred_element_type=jnp.float32)
    h = h + b_ref[...]
    h = jnp.tanh(h)
    h_ref[...] = h
    for j in range(8):
        hc_ref[j] = h[:, j * 16:(j + 1) * 16]
    cnt_ref[...] = cnt


_l1_call = pl.pallas_call(
    _l1_body,
    grid=(_N // _BN,),
    in_specs=[
        pl.BlockSpec((2, _BN, 16), lambda i: (0, i, 0)),
        pl.BlockSpec((_BN, 16), lambda i: (i, 0)),
        pl.BlockSpec((16, _H), lambda i: (0, 0)),
        pl.BlockSpec((16, _H), lambda i: (0, 0)),
        pl.BlockSpec((1, _H), lambda i: (0, 0)),
    ],
    out_specs=[
        pl.BlockSpec((_BN, _H), lambda i: (i, 0)),
        pl.BlockSpec((8, _BN, 16), lambda i: (0, i, 0)),
        pl.BlockSpec((_BN, 1), lambda i: (i, 0)),
    ],
    out_shape=[
        jax.ShapeDtypeStruct((_N, _H), jnp.float32),
        jax.ShapeDtypeStruct((8, _N2, 16), jnp.float32),
        jax.ShapeDtypeStruct((_N, 1), jnp.float32),
    ],
)


def _final_body(a2_ref, cnt_ref, h1_ref, wl2_ref, wr2_ref, bl2_ref,
                wlin_ref, blin_ref, wlast_ref, blast_ref, o_ref):
    summed = jnp.concatenate([a2_ref[j] for j in range(8)], axis=1)
    mean = summed / jnp.maximum(cnt_ref[...], 1.0)
    h2 = jnp.tanh(
        jnp.dot(mean, wl2_ref[...], preferred_element_type=jnp.float32)
        + jnp.dot(h1_ref[...], wr2_ref[...], preferred_element_type=jnp.float32)
        + bl2_ref[...])
    h3 = jnp.tanh(
        jnp.dot(h2, wlin_ref[...], preferred_element_type=jnp.float32)
        + blin_ref[...])
    lg = jnp.dot(h3, wlast_ref[...], preferred_element_type=jnp.float32)
    lg = lg + blast_ref[...]
    m = jnp.max(lg, axis=1, keepdims=True)
    e = jnp.exp(lg - m)
    o_ref[...] = e / jnp.sum(e, axis=1, keepdims=True)


_final_call = pl.pallas_call(
    _final_body,
    grid=(_N // _BN,),
    in_specs=[
        pl.BlockSpec((8, _BN, 16), lambda i: (0, i, 0)),
        pl.BlockSpec((_BN, 1), lambda i: (i, 0)),
        pl.BlockSpec((_BN, _H), lambda i: (i, 0)),
        pl.BlockSpec((_H, _H), lambda i: (0, 0)),
        pl.BlockSpec((_H, _H), lambda i: (0, 0)),
        pl.BlockSpec((1, _H), lambda i: (0, 0)),
        pl.BlockSpec((_H, _H), lambda i: (0, 0)),
        pl.BlockSpec((1, _H), lambda i: (0, 0)),
        pl.BlockSpec((_H, 4), lambda i: (0, 0)),
        pl.BlockSpec((1, 4), lambda i: (0, 0)),
    ],
    out_specs=pl.BlockSpec((_BN, 4), lambda i: (i, 0)),
    out_shape=jax.ShapeDtypeStruct((_N, 4), jnp.float32),
)


def kernel(x, edge_index, Wl1, bl1, Wr1, Wl2, bl2, Wr2,
           W_lin1, b_lin1, W_last, b_last):
    src = edge_index[0]
    dst = edge_index[1]
    pad = _EP + _B - _E
    srcp = jnp.concatenate([src, jnp.zeros((pad,), jnp.int32)])
    dstp = jnp.concatenate([dst, jnp.full((pad,), _N, jnp.int32)])
    dstp = dstp.reshape((_EP + _B) // 128, 128)

    xtp = jnp.pad(x.T, ((0, 0), (0, _N2 - _N)))
    theta = jnp.float32(np.pi / 2)
    ctst = jnp.stack([jnp.cos(theta), jnp.sin(theta)]).reshape(1, 2)
    xn16 = _norm_call(xtp, ctst).T

    agg1 = _sc_l1(xn16, srcp, dstp)
    Wl1p = jnp.pad(Wl1, ((0, 13), (0, 0)))
    Wr1p = jnp.pad(Wr1, ((0, 13), (0, 0)))
    h1, h1c, cnt = _l1_call(agg1, xn16, Wl1p, Wr1p, bl1.reshape(1, _H))

    agg2 = _sc_l2(h1c.reshape(8 * _N2, 16), srcp, dstp)
    out = _final_call(agg2, cnt, h1, Wl2, Wr2, bl2.reshape(1, _H),
                      W_lin1, b_lin1.reshape(1, _H),
                      W_last, b_last.reshape(1, 4))
    return out
